# baseline (speedup 1.0000x reference)
import functools

import jax
import jax.numpy as jnp
from jax import lax
from jax.experimental import pallas as pl
from jax.experimental.pallas import tpu as pltpu
from jax.experimental.pallas import tpu_sc as plsc

B = 16
LQ = 576
C = 1024
H = 8
P = 4
L = 3
DH = C // H
LV = 3024
NPTS = L * P

NC = 2
NS = 16
NW = NC * NS
NROWS = B * LQ * H
RPW = NROWS // NW
G = 8
STEPS = RPW // G
IDXW = G * NPTS
BODY = 3
FLUSH_ROWS = BODY * G


def _mm_bias_kernel(x_ref, w_ref, b_ref, o_ref):
    x = x_ref[...].astype(jnp.bfloat16)
    acc = lax.dot_general(x, w_ref[...], (((1,), (0,)), ((), ())),
                          preferred_element_type=jnp.float32)
    o_ref[...] = acc + b_ref[...]


def _mm_bias(x, w_bf16, bias, bm):
    m, k = x.shape
    n = w_bf16.shape[1]
    grid = m // bm
    return pl.pallas_call(
        _mm_bias_kernel,
        grid=(grid,),
        in_specs=[
            pl.BlockSpec((bm, k), lambda i: (i, 0)),
            pl.BlockSpec((k, n), lambda i: (0, 0)),
            pl.BlockSpec((1, n), lambda i: (0, 0)),
        ],
        out_specs=pl.BlockSpec((bm, n), lambda i: (i, 0)),
        out_shape=jax.ShapeDtypeStruct((m, n), jnp.float32),
    )(x, w_bf16, bias.reshape(1, n))


_BM1 = 768
_HPAIR = 2


def _vproj_kernel(x_ref, w_ref, b_ref, o_ref):
    x = x_ref[...].astype(jnp.bfloat16)
    acc = lax.dot_general(x, w_ref[...], (((1,), (0,)), ((), ())),
                          preferred_element_type=jnp.float32)
    acc = acc + b_ref[...]
    o_ref[0] = acc[:, 0:DH]
    o_ref[1] = acc[:, DH:2 * DH]


def _vproj_head_major(vflat, wv_bf16, bv):
    grid = (B * LV // _BM1, H // _HPAIR)
    return pl.pallas_call(
        _vproj_kernel,
        grid=grid,
        in_specs=[
            pl.BlockSpec((_BM1, C), lambda i, j: (i, 0)),
            pl.BlockSpec((C, _HPAIR * DH), lambda i, j: (0, j)),
            pl.BlockSpec((1, _HPAIR * DH), lambda i, j: (0, j)),
        ],
        out_specs=pl.BlockSpec((_HPAIR, _BM1, DH), lambda i, j: (j, i, 0)),
        out_shape=jax.ShapeDtypeStruct((H, B * LV, DH), jnp.float32),
    )(vflat, wv_bf16, bv.reshape(1, C))


_BM2 = 768


def _coord_kernel(q_ref, ws_ref, bsr_ref, wa_ref, ba_ref, fr_ref,
                  idx_ref, w_ref):
    i = pl.program_id(0)
    q = q_ref[...]
    offp = lax.dot_general(q, ws_ref[...], (((1,), (0,)), ((), ())),
                           precision=lax.Precision.DEFAULT,
                           preferred_element_type=jnp.float32) + bsr_ref[...]
    awp = lax.dot_general(q, wa_ref[...], (((1,), (0,)), ((), ())),
                          precision=lax.Precision.DEFAULT,
                          preferred_element_type=jnp.float32) + ba_ref[...]
    fx = fr_ref[:, 0:1]
    fy = fr_ref[:, 1:2]
    coordx = offp[:, 0:96] + fx
    coordy = offp[:, 96:192] + fy
    col = lax.broadcasted_iota(jnp.int32, (_BM2, 96), 1)
    lcol = (col // P) % L
    hcol = col // NPTS
    size = jnp.where(lcol == 0, 48, jnp.where(lcol == 1, 24, 12))
    lsi = jnp.where(lcol == 0, 0, jnp.where(lcol == 1, 2304, 2880))
    cx = jnp.clip(jnp.floor(coordx).astype(jnp.int32), 0, size - 1)
    cy = jnp.clip(jnp.floor(coordy).astype(jnp.int32), 0, size - 1)
    rowg = lax.broadcasted_iota(jnp.int32, (_BM2, 96), 0) + i * _BM2
    b = rowg // LQ
    idx_ref[...] = hcol * (B * LV) + b * LV + lsi + cx * size + cy
    for h in range(H):
        g = awp[:, h * NPTS:(h + 1) * NPTS]
        m = jnp.max(g, axis=1, keepdims=True)
        e = jnp.exp(g - m)
        w_ref[:, h * NPTS:(h + 1) * NPTS] = e / jnp.sum(e, axis=1, keepdims=True)


def _coords_and_weights(query2d, ws_perm, bs_perm, Wa, ba, final2d):
    m = query2d.shape[0]
    grid = m // _BM2
    return pl.pallas_call(
        _coord_kernel,
        grid=(grid,),
        in_specs=[
            pl.BlockSpec((_BM2, C), lambda i: (i, 0)),
            pl.BlockSpec((C, 192), lambda i: (0, 0)),
            pl.BlockSpec((1, 192), lambda i: (0, 0)),
            pl.BlockSpec((C, 96), lambda i: (0, 0)),
            pl.BlockSpec((1, 96), lambda i: (0, 0)),
            pl.BlockSpec((_BM2, 2), lambda i: (i, 0)),
        ],
        out_specs=[
            pl.BlockSpec((_BM2, 96), lambda i: (i, 0)),
            pl.BlockSpec((_BM2, 96), lambda i: (i, 0)),
        ],
        out_shape=[
            jax.ShapeDtypeStruct((m, 96), jnp.int32),
            jax.ShapeDtypeStruct((m, 96), jnp.float32),
        ],
    )(query2d, ws_perm, bs_perm.reshape(1, 192), Wa, ba.reshape(1, 96),
      final2d)


def _sc_gather_body(table_hbm, idx_hbm, w_hbm, out_hbm,
                    idx_v, w_v, rows_v, out_v,
                    gsem0, gsem1, gsem2, osem):
    wid = lax.axis_index("s") * NC + lax.axis_index("c")
    pltpu.sync_copy(idx_hbm.at[wid], idx_v)
    pltpu.sync_copy(w_hbm.at[wid], w_v)
    gsems = (gsem0, gsem1, gsem2)
    out_base = wid * RPW

    for k in range(BODY):
        pltpu.async_copy(table_hbm.at[idx_v.at[k]], rows_v.at[k], gsems[k])

    def body(s2, carry):
        @pl.when(s2 >= 1)
        def _wait_out():
            pltpu.make_async_copy(out_v,
                                  out_hbm.at[pl.ds(0, FLUSH_ROWS)],
                                  osem).wait()
        for k in range(BODY):
            s = s2 * BODY + k
            par = k
            pltpu.make_async_copy(table_hbm.at[idx_v.at[s]],
                                  rows_v.at[par], gsems[par]).wait()
            wvecs = [w_v[s, pl.ds(j * 16, 16)] for j in range(IDXW // 16)]
            for g in range(G):
                wsc = [wvecs[(g * NPTS + p) // 16][(g * NPTS + p) % 16]
                       for p in range(NPTS)]
                for c in range(DH // 16):
                    acc = wsc[0] * rows_v[par, g * NPTS, pl.ds(c * 16, 16)]
                    for p in range(1, NPTS):
                        acc = acc + wsc[p] * rows_v[par, g * NPTS + p,
                                                    pl.ds(c * 16, 16)]
                    out_v[k * G + g, pl.ds(c * 16, 16)] = acc
            @pl.when(s < STEPS - BODY)
            def _prefetch(s=s, par=par):
                pltpu.async_copy(table_hbm.at[idx_v.at[s + BODY]],
                                 rows_v.at[par], gsems[par])
        row0 = out_base + s2 * FLUSH_ROWS
        pltpu.async_copy(out_v, out_hbm.at[pl.ds(row0, FLUSH_ROWS)], osem)
        return carry

    lax.fori_loop(0, STEPS // BODY, body, 0)
    pltpu.make_async_copy(out_v, out_hbm.at[pl.ds(0, FLUSH_ROWS)],
                          osem).wait()


def _sc_gather(table, idx3, w3):
    mesh = plsc.VectorSubcoreMesh(core_axis_name="c", subcore_axis_name="s")
    f = functools.partial(
        pl.kernel,
        out_type=jax.ShapeDtypeStruct((NROWS, DH), jnp.float32),
        mesh=mesh,
        scratch_types=[
            pltpu.VMEM((STEPS, IDXW), jnp.int32),
            pltpu.VMEM((STEPS, IDXW), jnp.float32),
            pltpu.VMEM((BODY, IDXW, DH), jnp.float32),
            pltpu.VMEM((FLUSH_ROWS, DH), jnp.float32),
            pltpu.SemaphoreType.DMA,
            pltpu.SemaphoreType.DMA,
            pltpu.SemaphoreType.DMA,
            pltpu.SemaphoreType.DMA,
        ],
    )(_sc_gather_body)
    return f(table, idx3, w3)


def kernel(query, reference_points, value, spatial_shapes, level_start_index,
           Wv, bv, Ws, bs, Wa, ba, Wo, bo):
    ref = reference_points.reshape(B, LV, 1, 2)
    part1 = ref[:, :2304].reshape(B, 48, 48, 1, 2).mean(axis=(1, 2))
    part1 = jnp.broadcast_to(part1.reshape(B, 1, 1, 2), (B, LQ, 1, 2))
    part2 = ref[:, 2304:2880].reshape(B, LQ, 1, 2)
    p3 = ref[:, 2880:].reshape(B, 12, 12, 2).transpose(0, 3, 1, 2)
    p3 = jax.image.resize(p3, (B, 2, 24, 24), method="bilinear")
    part3 = p3.reshape(B, 2, LQ).transpose(0, 2, 1).reshape(B, LQ, 1, 2)
    final2d = ((part1 + part2 + part3) / 3.0).reshape(B * LQ, 2)

    vproj = _vproj_head_major(value.reshape(B * LV, C),
                              Wv.astype(jnp.bfloat16), bv)
    table = vproj.reshape(H * B * LV, DH)

    ws_perm = Ws.reshape(C, H, L, P, 2).transpose(0, 4, 1, 2, 3).reshape(C, 192)
    bs_perm = bs.reshape(H, L, P, 2).transpose(3, 0, 1, 2).reshape(192)
    idx, w = _coords_and_weights(query.reshape(B * LQ, C), ws_perm, bs_perm,
                                 Wa, ba, final2d)
    idx3 = idx.reshape(NW, STEPS, IDXW)
    w3 = w.reshape(NW, STEPS, IDXW)

    out = _sc_gather(table, idx3, w3)

    res = _mm_bias(out.reshape(B * LQ, C), Wo.astype(jnp.bfloat16), bo,
                   bm=768)
    return res.reshape(B, LQ, C)


# speedup vs baseline: 1.1510x; 1.1510x over previous
import functools

import jax
import jax.numpy as jnp
from jax import lax
from jax.experimental import pallas as pl
from jax.experimental.pallas import tpu as pltpu
from jax.experimental.pallas import tpu_sc as plsc

B = 16
LQ = 576
C = 1024
H = 8
P = 4
L = 3
DH = C // H
LV = 3024
NPTS = L * P

NC = 2
NS = 16
NW = NC * NS
NROWS = B * LQ * H
RPW = NROWS // NW
G = 8
STEPS = RPW // G
IDXW = G * NPTS
BODY = 2
FLUSH_ROWS = BODY * G


def _mm_bias_kernel(x_ref, w_ref, b_ref, o_ref):
    x = x_ref[...].astype(jnp.bfloat16)
    acc = lax.dot_general(x, w_ref[...], (((1,), (0,)), ((), ())),
                          preferred_element_type=jnp.float32)
    o_ref[...] = acc + b_ref[...]


def _mm_bias(x, w_bf16, bias, bm):
    m, k = x.shape
    n = w_bf16.shape[1]
    grid = m // bm
    return pl.pallas_call(
        _mm_bias_kernel,
        grid=(grid,),
        in_specs=[
            pl.BlockSpec((bm, k), lambda i: (i, 0)),
            pl.BlockSpec((k, n), lambda i: (0, 0)),
            pl.BlockSpec((1, n), lambda i: (0, 0)),
        ],
        out_specs=pl.BlockSpec((bm, n), lambda i: (i, 0)),
        out_shape=jax.ShapeDtypeStruct((m, n), jnp.float32),
    )(x, w_bf16, bias.reshape(1, n))


_BM1 = 768


def _vproj_kernel(x_ref, w_ref, b_ref, o_ref):
    x = x_ref[...].astype(jnp.bfloat16)
    acc = lax.dot_general(x, w_ref[...], (((1,), (0,)), ((), ())),
                          preferred_element_type=jnp.float32)
    acc = acc + b_ref[...]
    for h in range(H):
        o_ref[h] = acc[:, h * DH:(h + 1) * DH]


def _vproj_head_major(vflat, wv_bf16, bv):
    grid = (B * LV // _BM1,)
    return pl.pallas_call(
        _vproj_kernel,
        grid=grid,
        in_specs=[
            pl.BlockSpec((_BM1, C), lambda i: (i, 0)),
            pl.BlockSpec((C, C), lambda i: (0, 0)),
            pl.BlockSpec((1, C), lambda i: (0, 0)),
        ],
        out_specs=pl.BlockSpec((H, _BM1, DH), lambda i: (0, i, 0)),
        out_shape=jax.ShapeDtypeStruct((H, B * LV, DH), jnp.float32),
    )(vflat, wv_bf16, bv.reshape(1, C))


_BM2 = 768


def _coord_kernel(q_ref, ws_ref, bsr_ref, wa_ref, ba_ref, fr_ref,
                  idx_ref, w_ref):
    i = pl.program_id(0)
    q = q_ref[...]
    offp = lax.dot_general(q, ws_ref[...], (((1,), (0,)), ((), ())),
                           precision=lax.Precision.DEFAULT,
                           preferred_element_type=jnp.float32) + bsr_ref[...]
    awp = lax.dot_general(q, wa_ref[...], (((1,), (0,)), ((), ())),
                          precision=lax.Precision.DEFAULT,
                          preferred_element_type=jnp.float32) + ba_ref[...]
    fx = fr_ref[:, 0:1]
    fy = fr_ref[:, 1:2]
    coordx = offp[:, 0:96] + fx
    coordy = offp[:, 96:192] + fy
    col = lax.broadcasted_iota(jnp.int32, (_BM2, 96), 1)
    lcol = (col // P) % L
    hcol = col // NPTS
    size = jnp.where(lcol == 0, 48, jnp.where(lcol == 1, 24, 12))
    lsi = jnp.where(lcol == 0, 0, jnp.where(lcol == 1, 2304, 2880))
    cx = jnp.clip(jnp.floor(coordx).astype(jnp.int32), 0, size - 1)
    cy = jnp.clip(jnp.floor(coordy).astype(jnp.int32), 0, size - 1)
    rowg = lax.broadcasted_iota(jnp.int32, (_BM2, 96), 0) + i * _BM2
    b = rowg // LQ
    idx_ref[...] = hcol * (B * LV) + b * LV + lsi + cx * size + cy
    for h in range(H):
        g = awp[:, h * NPTS:(h + 1) * NPTS]
        m = jnp.max(g, axis=1, keepdims=True)
        e = jnp.exp(g - m)
        w_ref[:, h * NPTS:(h + 1) * NPTS] = e / jnp.sum(e, axis=1, keepdims=True)


def _coords_and_weights(query2d, ws_perm, bs_perm, Wa, ba, final2d):
    m = query2d.shape[0]
    grid = m // _BM2
    return pl.pallas_call(
        _coord_kernel,
        grid=(grid,),
        in_specs=[
            pl.BlockSpec((_BM2, C), lambda i: (i, 0)),
            pl.BlockSpec((C, 192), lambda i: (0, 0)),
            pl.BlockSpec((1, 192), lambda i: (0, 0)),
            pl.BlockSpec((C, 96), lambda i: (0, 0)),
            pl.BlockSpec((1, 96), lambda i: (0, 0)),
            pl.BlockSpec((_BM2, 2), lambda i: (i, 0)),
        ],
        out_specs=[
            pl.BlockSpec((_BM2, 96), lambda i: (i, 0)),
            pl.BlockSpec((_BM2, 96), lambda i: (i, 0)),
        ],
        out_shape=[
            jax.ShapeDtypeStruct((m, 96), jnp.int32),
            jax.ShapeDtypeStruct((m, 96), jnp.float32),
        ],
    )(query2d, ws_perm, bs_perm.reshape(1, 192), Wa, ba.reshape(1, 96),
      final2d)


def _sc_gather_body(table_hbm, idx_hbm, w_hbm, out_hbm,
                    idx_v, w_v, rows_v, out_v,
                    gsem0, gsem1, osem):
    wid = lax.axis_index("s") * NC + lax.axis_index("c")
    pltpu.sync_copy(idx_hbm.at[wid], idx_v)
    pltpu.sync_copy(w_hbm.at[wid], w_v)
    gsems = (gsem0, gsem1)
    out_base = wid * RPW

    for k in range(BODY):
        pltpu.async_copy(table_hbm.at[idx_v.at[k]], rows_v.at[k], gsems[k])

    def body(s2, carry):
        @pl.when(s2 >= 1)
        def _wait_out():
            pltpu.make_async_copy(out_v,
                                  out_hbm.at[pl.ds(0, FLUSH_ROWS)],
                                  osem).wait()
        for k in range(BODY):
            s = s2 * BODY + k
            par = k
            pltpu.make_async_copy(table_hbm.at[idx_v.at[s]],
                                  rows_v.at[par], gsems[par]).wait()
            wvecs = [w_v[s, pl.ds(j * 16, 16)] for j in range(IDXW // 16)]
            for g in range(G):
                wsc = [wvecs[(g * NPTS + p) // 16][(g * NPTS + p) % 16]
                       for p in range(NPTS)]
                for c in range(DH // 16):
                    acc = wsc[0] * rows_v[par, g * NPTS, pl.ds(c * 16, 16)]
                    for p in range(1, NPTS):
                        acc = acc + wsc[p] * rows_v[par, g * NPTS + p,
                                                    pl.ds(c * 16, 16)]
                    out_v[k * G + g, pl.ds(c * 16, 16)] = acc
            @pl.when(s < STEPS - BODY)
            def _prefetch(s=s, par=par):
                pltpu.async_copy(table_hbm.at[idx_v.at[s + BODY]],
                                 rows_v.at[par], gsems[par])
        row0 = out_base + s2 * FLUSH_ROWS
        pltpu.async_copy(out_v, out_hbm.at[pl.ds(row0, FLUSH_ROWS)], osem)
        return carry

    lax.fori_loop(0, STEPS // BODY, body, 0)
    pltpu.make_async_copy(out_v, out_hbm.at[pl.ds(0, FLUSH_ROWS)],
                          osem).wait()


def _sc_gather(table, idx3, w3):
    mesh = plsc.VectorSubcoreMesh(core_axis_name="c", subcore_axis_name="s")
    f = functools.partial(
        pl.kernel,
        out_type=jax.ShapeDtypeStruct((NROWS, DH), jnp.float32),
        mesh=mesh,
        scratch_types=[
            pltpu.VMEM((STEPS, IDXW), jnp.int32),
            pltpu.VMEM((STEPS, IDXW), jnp.float32),
            pltpu.VMEM((BODY, IDXW, DH), jnp.float32),
            pltpu.VMEM((FLUSH_ROWS, DH), jnp.float32),
            pltpu.SemaphoreType.DMA,
            pltpu.SemaphoreType.DMA,
            pltpu.SemaphoreType.DMA,
        ],
    )(_sc_gather_body)
    return f(table, idx3, w3)


def kernel(query, reference_points, value, spatial_shapes, level_start_index,
           Wv, bv, Ws, bs, Wa, ba, Wo, bo):
    ref = reference_points.reshape(B, LV, 1, 2)
    part1 = ref[:, :2304].reshape(B, 48, 48, 1, 2).mean(axis=(1, 2))
    part1 = jnp.broadcast_to(part1.reshape(B, 1, 1, 2), (B, LQ, 1, 2))
    part2 = ref[:, 2304:2880].reshape(B, LQ, 1, 2)
    p3 = ref[:, 2880:].reshape(B, 12, 12, 2).transpose(0, 3, 1, 2)
    p3 = jax.image.resize(p3, (B, 2, 24, 24), method="bilinear")
    part3 = p3.reshape(B, 2, LQ).transpose(0, 2, 1).reshape(B, LQ, 1, 2)
    final2d = ((part1 + part2 + part3) / 3.0).reshape(B * LQ, 2)

    vproj = _vproj_head_major(value.reshape(B * LV, C),
                              Wv.astype(jnp.bfloat16), bv)
    table = vproj.reshape(H * B * LV, DH)

    ws_perm = Ws.reshape(C, H, L, P, 2).transpose(0, 4, 1, 2, 3).reshape(C, 192)
    bs_perm = bs.reshape(H, L, P, 2).transpose(3, 0, 1, 2).reshape(192)
    idx, w = _coords_and_weights(query.reshape(B * LQ, C), ws_perm, bs_perm,
                                 Wa, ba, final2d)
    idx3 = idx.reshape(NW, STEPS, IDXW)
    w3 = w.reshape(NW, STEPS, IDXW)

    out = _sc_gather(table, idx3, w3)

    res = _mm_bias(out.reshape(B * LQ, C), Wo.astype(jnp.bfloat16), bo,
                   bm=768)
    return res.reshape(B, LQ, C)


# speedup vs baseline: 1.1542x; 1.0028x over previous
import functools

import jax
import jax.numpy as jnp
from jax import lax
from jax.experimental import pallas as pl
from jax.experimental.pallas import tpu as pltpu
from jax.experimental.pallas import tpu_sc as plsc

B = 16
LQ = 576
C = 1024
H = 8
P = 4
L = 3
DH = C // H
LV = 3024
NPTS = L * P

NC = 2
NS = 16
NW = NC * NS
NROWS = B * LQ * H
RPW = NROWS // NW
G = 8
STEPS = RPW // G
IDXW = G * NPTS
BODY = 2
FLUSH_ROWS = BODY * G


def _mm_bias_kernel(x_ref, w_ref, b_ref, o_ref):
    x = x_ref[...].astype(jnp.bfloat16)
    acc = lax.dot_general(x, w_ref[...], (((1,), (0,)), ((), ())),
                          preferred_element_type=jnp.float32)
    o_ref[...] = acc + b_ref[...]


def _mm_bias(x, w_bf16, bias, bm):
    m, k = x.shape
    n = w_bf16.shape[1]
    grid = m // bm
    return pl.pallas_call(
        _mm_bias_kernel,
        grid=(grid,),
        in_specs=[
            pl.BlockSpec((bm, k), lambda i: (i, 0)),
            pl.BlockSpec((k, n), lambda i: (0, 0)),
            pl.BlockSpec((1, n), lambda i: (0, 0)),
        ],
        out_specs=pl.BlockSpec((bm, n), lambda i: (i, 0)),
        out_shape=jax.ShapeDtypeStruct((m, n), jnp.float32),
    )(x, w_bf16, bias.reshape(1, n))


_BM1 = 768


def _vproj_kernel(x_ref, w_ref, b_ref, o_ref):
    x = x_ref[...].astype(jnp.bfloat16)
    acc = lax.dot_general(x, w_ref[...], (((1,), (0,)), ((), ())),
                          preferred_element_type=jnp.float32)
    acc = acc + b_ref[...]
    for h in range(H):
        o_ref[h] = acc[:, h * DH:(h + 1) * DH]


def _vproj_head_major(vflat, wv_bf16, bv):
    grid = (B * LV // _BM1,)
    return pl.pallas_call(
        _vproj_kernel,
        grid=grid,
        in_specs=[
            pl.BlockSpec((_BM1, C), lambda i: (i, 0)),
            pl.BlockSpec((C, C), lambda i: (0, 0)),
            pl.BlockSpec((1, C), lambda i: (0, 0)),
        ],
        out_specs=pl.BlockSpec((H, _BM1, DH), lambda i: (0, i, 0)),
        out_shape=jax.ShapeDtypeStruct((H, B * LV, DH), jnp.float32),
    )(vflat, wv_bf16, bv.reshape(1, C))


_BM2 = 768


def _coord_kernel(q_ref, ws_ref, bsr_ref, wa_ref, ba_ref, fr_ref,
                  idx_ref, w_ref):
    i = pl.program_id(0)
    q = q_ref[...]
    offp = lax.dot_general(q, ws_ref[...], (((1,), (0,)), ((), ())),
                           precision=lax.Precision.DEFAULT,
                           preferred_element_type=jnp.float32) + bsr_ref[...]
    awp = lax.dot_general(q, wa_ref[...], (((1,), (0,)), ((), ())),
                          precision=lax.Precision.DEFAULT,
                          preferred_element_type=jnp.float32) + ba_ref[...]
    fx = fr_ref[:, 0:1]
    fy = fr_ref[:, 1:2]
    coordx = offp[:, 0:96] + fx
    coordy = offp[:, 96:192] + fy
    col = lax.broadcasted_iota(jnp.int32, (_BM2, 96), 1)
    lcol = (col // P) % L
    hcol = col // NPTS
    size = jnp.where(lcol == 0, 48, jnp.where(lcol == 1, 24, 12))
    lsi = jnp.where(lcol == 0, 0, jnp.where(lcol == 1, 2304, 2880))
    cx = jnp.clip(jnp.floor(coordx).astype(jnp.int32), 0, size - 1)
    cy = jnp.clip(jnp.floor(coordy).astype(jnp.int32), 0, size - 1)
    rowg = lax.broadcasted_iota(jnp.int32, (_BM2, 96), 0) + i * _BM2
    b = rowg // LQ
    idx_ref[...] = hcol * (B * LV) + b * LV + lsi + cx * size + cy
    for h in range(H):
        g = awp[:, h * NPTS:(h + 1) * NPTS]
        m = jnp.max(g, axis=1, keepdims=True)
        e = jnp.exp(g - m)
        w_ref[:, h * NPTS:(h + 1) * NPTS] = e / jnp.sum(e, axis=1, keepdims=True)


def _coords_and_weights(query2d, ws_perm, bs_perm, Wa, ba, final2d):
    m = query2d.shape[0]
    grid = m // _BM2
    return pl.pallas_call(
        _coord_kernel,
        grid=(grid,),
        in_specs=[
            pl.BlockSpec((_BM2, C), lambda i: (i, 0)),
            pl.BlockSpec((C, 192), lambda i: (0, 0)),
            pl.BlockSpec((1, 192), lambda i: (0, 0)),
            pl.BlockSpec((C, 96), lambda i: (0, 0)),
            pl.BlockSpec((1, 96), lambda i: (0, 0)),
            pl.BlockSpec((_BM2, 2), lambda i: (i, 0)),
        ],
        out_specs=[
            pl.BlockSpec((_BM2, 96), lambda i: (i, 0)),
            pl.BlockSpec((_BM2, 96), lambda i: (i, 0)),
        ],
        out_shape=[
            jax.ShapeDtypeStruct((m, 96), jnp.int32),
            jax.ShapeDtypeStruct((m, 96), jnp.float32),
        ],
    )(query2d, ws_perm, bs_perm.reshape(1, 192), Wa, ba.reshape(1, 96),
      final2d)


def _sc_gather_body(table_hbm, idx_hbm, w_hbm, out_hbm,
                    idx_v, w_v, rows_v, out_v,
                    gsem0, gsem1, osem):
    wid = lax.axis_index("s") * NC + lax.axis_index("c")
    pltpu.sync_copy(idx_hbm.at[wid], idx_v)
    pltpu.sync_copy(w_hbm.at[wid], w_v)
    gsems = (gsem0, gsem1)
    out_base = wid * RPW

    HALF = IDXW // 2

    def _start_gather(s, slot):
        pltpu.async_copy(table_hbm.at[idx_v.at[s, pl.ds(0, HALF)]],
                         rows_v.at[slot, pl.ds(0, HALF)], gsems[slot])
        pltpu.async_copy(table_hbm.at[idx_v.at[s, pl.ds(HALF, HALF)]],
                         rows_v.at[slot, pl.ds(HALF, HALF)], gsems[slot])

    def _wait_gather(slot):
        pltpu.make_async_copy(table_hbm.at[idx_v.at[0, pl.ds(0, HALF)]],
                              rows_v.at[slot, pl.ds(0, HALF)],
                              gsems[slot]).wait()
        pltpu.make_async_copy(table_hbm.at[idx_v.at[0, pl.ds(0, HALF)]],
                              rows_v.at[slot, pl.ds(HALF, HALF)],
                              gsems[slot]).wait()

    for k in range(BODY):
        _start_gather(k, k)

    def body(s2, carry):
        @pl.when(s2 >= 1)
        def _wait_out():
            pltpu.make_async_copy(out_v,
                                  out_hbm.at[pl.ds(0, FLUSH_ROWS)],
                                  osem).wait()
        for k in range(BODY):
            s = s2 * BODY + k
            par = k
            _wait_gather(par)
            wvecs = [w_v[s, pl.ds(j * 16, 16)] for j in range(IDXW // 16)]
            for g in range(G):
                wsc = [wvecs[(g * NPTS + p) // 16][(g * NPTS + p) % 16]
                       for p in range(NPTS)]
                for c in range(DH // 16):
                    acc = wsc[0] * rows_v[par, g * NPTS, pl.ds(c * 16, 16)]
                    for p in range(1, NPTS):
                        acc = acc + wsc[p] * rows_v[par, g * NPTS + p,
                                                    pl.ds(c * 16, 16)]
                    out_v[k * G + g, pl.ds(c * 16, 16)] = acc
            @pl.when(s < STEPS - BODY)
            def _prefetch(s=s, par=par):
                _start_gather(s + BODY, par)
        row0 = out_base + s2 * FLUSH_ROWS
        pltpu.async_copy(out_v, out_hbm.at[pl.ds(row0, FLUSH_ROWS)], osem)
        return carry

    lax.fori_loop(0, STEPS // BODY, body, 0)
    pltpu.make_async_copy(out_v, out_hbm.at[pl.ds(0, FLUSH_ROWS)],
                          osem).wait()


def _sc_gather(table, idx3, w3):
    mesh = plsc.VectorSubcoreMesh(core_axis_name="c", subcore_axis_name="s")
    f = functools.partial(
        pl.kernel,
        out_type=jax.ShapeDtypeStruct((NROWS, DH), jnp.float32),
        mesh=mesh,
        scratch_types=[
            pltpu.VMEM((STEPS, IDXW), jnp.int32),
            pltpu.VMEM((STEPS, IDXW), jnp.float32),
            pltpu.VMEM((BODY, IDXW, DH), jnp.float32),
            pltpu.VMEM((FLUSH_ROWS, DH), jnp.float32),
            pltpu.SemaphoreType.DMA,
            pltpu.SemaphoreType.DMA,
            pltpu.SemaphoreType.DMA,
        ],
    )(_sc_gather_body)
    return f(table, idx3, w3)


def kernel(query, reference_points, value, spatial_shapes, level_start_index,
           Wv, bv, Ws, bs, Wa, ba, Wo, bo):
    ref = reference_points.reshape(B, LV, 1, 2)
    part1 = ref[:, :2304].reshape(B, 48, 48, 1, 2).mean(axis=(1, 2))
    part1 = jnp.broadcast_to(part1.reshape(B, 1, 1, 2), (B, LQ, 1, 2))
    part2 = ref[:, 2304:2880].reshape(B, LQ, 1, 2)
    p3 = ref[:, 2880:].reshape(B, 12, 12, 2).transpose(0, 3, 1, 2)
    p3 = jax.image.resize(p3, (B, 2, 24, 24), method="bilinear")
    part3 = p3.reshape(B, 2, LQ).transpose(0, 2, 1).reshape(B, LQ, 1, 2)
    final2d = ((part1 + part2 + part3) / 3.0).reshape(B * LQ, 2)

    vproj = _vproj_head_major(value.reshape(B * LV, C),
                              Wv.astype(jnp.bfloat16), bv)
    table = vproj.reshape(H * B * LV, DH)

    ws_perm = Ws.reshape(C, H, L, P, 2).transpose(0, 4, 1, 2, 3).reshape(C, 192)
    bs_perm = bs.reshape(H, L, P, 2).transpose(3, 0, 1, 2).reshape(192)
    idx, w = _coords_and_weights(query.reshape(B * LQ, C), ws_perm, bs_perm,
                                 Wa, ba, final2d)
    idx3 = idx.reshape(NW, STEPS, IDXW)
    w3 = w.reshape(NW, STEPS, IDXW)

    out = _sc_gather(table, idx3, w3)

    res = _mm_bias(out.reshape(B * LQ, C), Wo.astype(jnp.bfloat16), bo,
                   bm=768)
    return res.reshape(B, LQ, C)
